# baseline (speedup 1.0000x reference)
import jax
import jax.numpy as jnp
from jax.experimental import pallas as pl
from jax.experimental.pallas import tpu as pltpu

H = 4
C = 64


def _silu(v):
    return v * jax.nn.sigmoid(v)


def _final_block(out_ref, gacc_ref, x_ref, bias_ref, ln_w_ref, ln_b_ref, o_ref):
    out = out_ref[...] + bias_ref[...]
    gsum = gacc_ref[:, 0:1]
    deg = jnp.maximum(gacc_ref[:, 1:2], 1.0)
    out = out * (gsum / deg)
    mu = jnp.mean(out, axis=-1, keepdims=True)
    var = jnp.mean((out - mu) ** 2, axis=-1, keepdims=True)
    out = (out - mu) * jax.lax.rsqrt(var + 1e-5) * ln_w_ref[...] + ln_b_ref[...]
    out = out * jax.nn.sigmoid(out)
    o_ref[...] = out + x_ref[...]


def kernel(x, edge_index, edge_attr, W_l, b_l, W_r, b_r, W_e, att, bias,
           eg_W1, eg_b1, eg_W2, eg_b2, ln_w, ln_b):
    src = edge_index[0]
    dst = edge_index[1]
    n = x.shape[0]
    x_l = (x @ W_l + b_l).reshape(n, H, C)
    x_r = (x @ W_r + b_r).reshape(n, H, C)
    e_emb = (edge_attr @ W_e).reshape(-1, H, C)
    e = x_l[src] + x_r[dst] + e_emb
    e = jax.nn.leaky_relu(e, 0.2)
    alpha = jnp.sum(e * att[None, :, :], axis=-1)
    m = jax.ops.segment_max(alpha, dst, num_segments=n)
    a = jnp.exp(alpha - m[dst])
    denom = jax.ops.segment_sum(a, dst, num_segments=n)
    alpha_n = a / (denom[dst] + 1e-16)
    msg = x_l[src] * alpha_n[:, :, None]
    out = jax.ops.segment_sum(msg, dst, num_segments=n).reshape(n, H * C)
    gate = jax.nn.sigmoid(_silu(edge_attr @ eg_W1 + eg_b1) @ eg_W2 + eg_b2)
    gacc = jnp.concatenate([gate, jnp.ones_like(gate)], axis=1)
    gacc = jax.ops.segment_sum(gacc, dst, num_segments=n)

    BN = 1000
    return pl.pallas_call(
        _final_block,
        out_shape=jax.ShapeDtypeStruct((n, H * C), jnp.float32),
        grid=(n // BN,),
        in_specs=[
            pl.BlockSpec((BN, H * C), lambda i: (i, 0)),
            pl.BlockSpec((BN, 2), lambda i: (i, 0)),
            pl.BlockSpec((BN, H * C), lambda i: (i, 0)),
            pl.BlockSpec((1, H * C), lambda i: (0, 0)),
            pl.BlockSpec((1, H * C), lambda i: (0, 0)),
            pl.BlockSpec((1, H * C), lambda i: (0, 0)),
        ],
        out_specs=pl.BlockSpec((BN, H * C), lambda i: (i, 0)),
    )(out, gacc, x, bias.reshape(1, H * C), ln_w.reshape(1, H * C),
      ln_b.reshape(1, H * C))


# speedup vs baseline: 11.3794x; 11.3794x over previous
import functools

import jax
import jax.numpy as jnp
from jax import lax
from jax.experimental import pallas as pl
from jax.experimental.pallas import tpu as pltpu
from jax.experimental.pallas import tpu_sc as plsc

H = 4
C = 64
HC = H * C
ED = 16
AW = 128
G = 128
NTILES = 32


def _p1_body(x_ref, wl_ref, bl_ref, wr_ref, br_ref, xl_ref, xr_ref):
    xb = x_ref[...]
    xl_ref[...] = jnp.dot(xb, wl_ref[...], preferred_element_type=jnp.float32) + bl_ref[...]
    xr_ref[...] = jnp.dot(xb, wr_ref[...], preferred_element_type=jnp.float32) + br_ref[...]


def _make_gather(e_pad):
    per_tile = e_pad // NTILES
    iters = per_tile // G
    mesh = plsc.VectorSubcoreMesh(core_axis_name="c", subcore_axis_name="s")

    @functools.partial(
        pl.kernel,
        mesh=mesh,
        out_type=[jax.ShapeDtypeStruct((e_pad, HC), jnp.float32),
                  jax.ShapeDtypeStruct((e_pad, HC), jnp.float32)],
        scratch_types=[pltpu.VMEM((G,), jnp.int32),
                       pltpu.VMEM((G,), jnp.int32),
                       pltpu.VMEM((G, HC), jnp.float32),
                       pltpu.VMEM((G, HC), jnp.float32)],
    )
    def gather_k(xl_hbm, xr_hbm, src_hbm, dst_hbm, ol_hbm, or_hbm,
                 si_v, di_v, rl_v, rr_v):
        wid = lax.axis_index("s") * 2 + lax.axis_index("c")
        base = wid * per_tile

        @pl.loop(0, iters)
        def _(g):
            off = base + g * G
            pltpu.sync_copy(src_hbm.at[pl.ds(off, G)], si_v)
            pltpu.sync_copy(dst_hbm.at[pl.ds(off, G)], di_v)
            pltpu.sync_copy(xl_hbm.at[si_v], rl_v)
            pltpu.sync_copy(xr_hbm.at[di_v], rr_v)
            pltpu.sync_copy(rl_v, ol_hbm.at[pl.ds(off, G)])
            pltpu.sync_copy(rr_v, or_hbm.at[pl.ds(off, G)])

    return gather_k


def _p3_body(xl_ref, xr_ref, ea_ref, we_ref, att_ref, w1_ref, b1_ref,
             w2_ref, b2_ref, aux_ref, m_ref, m_acc):
    i = pl.program_id(0)
    ee = jnp.dot(ea_ref[...], we_ref[...], preferred_element_type=jnp.float32)
    v = xl_ref[...] + xr_ref[...] + ee
    v = jnp.where(v >= 0, v, 0.2 * v)
    vm = v * att_ref[...]
    cols = [jnp.sum(vm[:, h * C:(h + 1) * C], axis=1, keepdims=True)
            for h in range(H)]
    alpha = jnp.concatenate(cols, axis=1)
    g1 = jnp.dot(ea_ref[...], w1_ref[...], preferred_element_type=jnp.float32) + b1_ref[...]
    g1 = g1 * jax.nn.sigmoid(g1)
    g2 = jnp.dot(g1, w2_ref[...], preferred_element_type=jnp.float32) + b2_ref[...]
    gate = jax.nn.sigmoid(g2)
    one = jnp.ones_like(gate)
    zero = jnp.zeros_like(gate)
    aux_ref[...] = jnp.concatenate([alpha, gate, one, zero, zero], axis=1)
    blkmax = jnp.max(alpha)

    @pl.when(i == 0)
    def _():
        m_acc[0, 0] = blkmax

    @pl.when(i > 0)
    def _():
        m_acc[0, 0] = jnp.maximum(m_acc[0, 0], blkmax)

    m_ref[...] = jnp.full((1, 1), m_acc[0, 0], jnp.float32)


def _p4_body(e_real, be, aux_ref, xl_ref, m_ref, msg0_ref, msg1_ref, den_ref):
    i = pl.program_id(0)
    mglob = m_ref[...]
    aux = aux_ref[...]
    xl = xl_ref[...]
    rows = i * be + lax.broadcasted_iota(jnp.int32, (be, 1), 0)
    valid = (rows < e_real).astype(jnp.float32)
    a = jnp.exp(aux[:, 0:4] - mglob) * valid
    gate = aux[:, 4:5] * valid
    one = aux[:, 5:6] * valid
    msg0_ref[...] = jnp.concatenate(
        [xl[:, 0:64] * a[:, 0:1], xl[:, 64:128] * a[:, 1:2]], axis=1)
    msg1_ref[...] = jnp.concatenate(
        [xl[:, 128:192] * a[:, 2:3], xl[:, 192:256] * a[:, 3:4]], axis=1)
    den_ref[...] = jnp.concatenate(
        [a, gate, one, jnp.zeros((be, AW - 6), jnp.float32)], axis=1)


def _make_scatter(n_nodes, e_pad):
    per_sub = e_pad // 16
    iters = per_sub // G
    mesh = plsc.VectorSubcoreMesh(core_axis_name="c", subcore_axis_name="s")

    @functools.partial(
        pl.kernel,
        mesh=mesh,
        out_type=[jax.ShapeDtypeStruct((n_nodes, AW), jnp.float32),
                  jax.ShapeDtypeStruct((n_nodes, AW), jnp.float32)],
        scratch_types=[pltpu.VMEM((G, AW), jnp.float32),
                       pltpu.VMEM((G,), jnp.int32),
                       pltpu.VMEM_SHARED((n_nodes, AW), jnp.float32)],
    )
    def scatter_k(msg0_hbm, msg1_hbm, dst_hbm, zeros_hbm, o0_hbm, o1_hbm,
                  buf_v, idx_v, table_sh):
        cid = lax.axis_index("c")
        sid = lax.axis_index("s")

        @pl.when(sid == 0)
        def _():
            pltpu.sync_copy(zeros_hbm, table_sh)

        plsc.subcore_barrier()

        def stream(msg_hbm):
            @pl.loop(0, iters)
            def _(g):
                off = sid * per_sub + g * G
                pltpu.sync_copy(dst_hbm.at[pl.ds(off, G)], idx_v)
                pltpu.sync_copy(msg_hbm.at[pl.ds(off, G)], buf_v)
                pltpu.sync_copy(buf_v, table_sh.at[idx_v], add=True)

        @pl.when(cid == 0)
        def _():
            stream(msg0_hbm)

        @pl.when(cid == 1)
        def _():
            stream(msg1_hbm)

        plsc.subcore_barrier()

        @pl.when((sid == 0) & (cid == 0))
        def _():
            pltpu.sync_copy(table_sh, o0_hbm)

        @pl.when((sid == 0) & (cid == 1))
        def _():
            pltpu.sync_copy(table_sh, o1_hbm)

    return scatter_k


def _make_den_scatter(n_nodes, e_pad):
    per_sub = e_pad // NTILES
    iters = per_sub // G
    mesh = plsc.VectorSubcoreMesh(core_axis_name="c", subcore_axis_name="s")

    @functools.partial(
        pl.kernel,
        mesh=mesh,
        out_type=[jax.ShapeDtypeStruct((n_nodes, AW), jnp.float32),
                  jax.ShapeDtypeStruct((n_nodes, AW), jnp.float32)],
        scratch_types=[pltpu.VMEM((G, AW), jnp.float32),
                       pltpu.VMEM((G,), jnp.int32),
                       pltpu.VMEM_SHARED((n_nodes, AW), jnp.float32)],
    )
    def den_k(den_hbm, dst_hbm, zeros_hbm, o0_hbm, o1_hbm,
              buf_v, idx_v, table_sh):
        cid = lax.axis_index("c")
        sid = lax.axis_index("s")

        @pl.when(sid == 0)
        def _():
            pltpu.sync_copy(zeros_hbm, table_sh)

        plsc.subcore_barrier()

        @pl.loop(0, iters)
        def _(g):
            off = cid * (e_pad // 2) + sid * per_sub + g * G
            pltpu.sync_copy(dst_hbm.at[pl.ds(off, G)], idx_v)
            pltpu.sync_copy(den_hbm.at[pl.ds(off, G)], buf_v)
            pltpu.sync_copy(buf_v, table_sh.at[idx_v], add=True)

        plsc.subcore_barrier()

        @pl.when((sid == 0) & (cid == 0))
        def _():
            pltpu.sync_copy(table_sh, o0_hbm)

        @pl.when((sid == 0) & (cid == 1))
        def _():
            pltpu.sync_copy(table_sh, o1_hbm)

    return den_k


def _p6_body(acc0_ref, acc1_ref, den0_ref, den1_ref, x_ref, bias_ref,
             lnw_ref, lnb_ref, o_ref):
    a0 = acc0_ref[...]
    a1 = acc1_ref[...]
    bn = a0.shape[0]
    d8 = den0_ref[:, 0:8] + den1_ref[:, 0:8]
    num = jnp.concatenate([a0, a1], axis=1)
    den = jnp.concatenate(
        [jnp.broadcast_to(d8[:, h:h + 1], (bn, C)) for h in range(H)], axis=1)
    out = num / (den + 1e-30) + bias_ref[...]
    mean_gate = d8[:, 4:5] / jnp.maximum(d8[:, 5:6], 1.0)
    out = out * mean_gate
    mu = jnp.mean(out, axis=-1, keepdims=True)
    var = jnp.mean((out - mu) ** 2, axis=-1, keepdims=True)
    out = (out - mu) * lax.rsqrt(var + 1e-5) * lnw_ref[...] + lnb_ref[...]
    out = out * jax.nn.sigmoid(out)
    o_ref[...] = out + x_ref[...]


def kernel(x, edge_index, edge_attr, W_l, b_l, W_r, b_r, W_e, att, bias,
           eg_W1, eg_b1, eg_W2, eg_b2, ln_w, ln_b):
    n = x.shape[0]
    e = edge_attr.shape[0]
    e_pad = ((e + NTILES * G - 1) // (NTILES * G)) * (NTILES * G)
    pad = e_pad - e
    src_p = jnp.concatenate([edge_index[0].astype(jnp.int32),
                             jnp.zeros((pad,), jnp.int32)])
    dst_p = jnp.concatenate([edge_index[1].astype(jnp.int32),
                             jnp.zeros((pad,), jnp.int32)])
    ea_p = jnp.concatenate([edge_attr, jnp.zeros((pad, ED), edge_attr.dtype)])

    BN = 1000
    xl, xr = pl.pallas_call(
        _p1_body,
        out_shape=[jax.ShapeDtypeStruct((n, HC), jnp.float32)] * 2,
        grid=(n // BN,),
        in_specs=[
            pl.BlockSpec((BN, HC), lambda i: (i, 0)),
            pl.BlockSpec((HC, HC), lambda i: (0, 0)),
            pl.BlockSpec((1, HC), lambda i: (0, 0)),
            pl.BlockSpec((HC, HC), lambda i: (0, 0)),
            pl.BlockSpec((1, HC), lambda i: (0, 0)),
        ],
        out_specs=[pl.BlockSpec((BN, HC), lambda i: (i, 0))] * 2,
    )(x, W_l, b_l.reshape(1, HC), W_r, b_r.reshape(1, HC))

    xl_src, xr_dst = _make_gather(e_pad)(xl, xr, src_p, dst_p)

    BE = 2048
    n_eblk = e_pad // BE
    aux, mglob = pl.pallas_call(
        _p3_body,
        out_shape=[jax.ShapeDtypeStruct((e_pad, 8), jnp.float32),
                   jax.ShapeDtypeStruct((1, 1), jnp.float32)],
        grid=(n_eblk,),
        in_specs=[
            pl.BlockSpec((BE, HC), lambda i: (i, 0)),
            pl.BlockSpec((BE, HC), lambda i: (i, 0)),
            pl.BlockSpec((BE, ED), lambda i: (i, 0)),
            pl.BlockSpec((ED, HC), lambda i: (0, 0)),
            pl.BlockSpec((1, HC), lambda i: (0, 0)),
            pl.BlockSpec((ED, 2 * ED), lambda i: (0, 0)),
            pl.BlockSpec((1, 2 * ED), lambda i: (0, 0)),
            pl.BlockSpec((2 * ED, 1), lambda i: (0, 0)),
            pl.BlockSpec((1, 1), lambda i: (0, 0)),
        ],
        out_specs=[pl.BlockSpec((BE, 8), lambda i: (i, 0)),
                   pl.BlockSpec((1, 1), lambda i: (0, 0))],
        scratch_shapes=[pltpu.SMEM((1, 1), jnp.float32)],
    )(xl_src, xr_dst, ea_p, W_e, att.reshape(1, HC), eg_W1,
      eg_b1.reshape(1, 2 * ED), eg_W2, eg_b2.reshape(1, 1))

    msg0, msg1, denrows = pl.pallas_call(
        functools.partial(_p4_body, e, BE),
        out_shape=[jax.ShapeDtypeStruct((e_pad, AW), jnp.float32)] * 3,
        grid=(n_eblk,),
        in_specs=[
            pl.BlockSpec((BE, 8), lambda i: (i, 0)),
            pl.BlockSpec((BE, HC), lambda i: (i, 0)),
            pl.BlockSpec((1, 1), lambda i: (0, 0)),
        ],
        out_specs=[pl.BlockSpec((BE, AW), lambda i: (i, 0))] * 3,
    )(aux, xl_src, mglob)

    zeros_tbl = jnp.zeros((n, AW), jnp.float32)
    acc0, acc1 = _make_scatter(n, e_pad)(msg0, msg1, dst_p, zeros_tbl)
    den0, den1 = _make_den_scatter(n, e_pad)(denrows, dst_p, zeros_tbl)

    out = pl.pallas_call(
        _p6_body,
        out_shape=jax.ShapeDtypeStruct((n, HC), jnp.float32),
        grid=(n // BN,),
        in_specs=[
            pl.BlockSpec((BN, AW), lambda i: (i, 0)),
            pl.BlockSpec((BN, AW), lambda i: (i, 0)),
            pl.BlockSpec((BN, AW), lambda i: (i, 0)),
            pl.BlockSpec((BN, AW), lambda i: (i, 0)),
            pl.BlockSpec((BN, HC), lambda i: (i, 0)),
            pl.BlockSpec((1, HC), lambda i: (0, 0)),
            pl.BlockSpec((1, HC), lambda i: (0, 0)),
            pl.BlockSpec((1, HC), lambda i: (0, 0)),
        ],
        out_specs=pl.BlockSpec((BN, HC), lambda i: (i, 0)),
    )(acc0, acc1, den0, den1, x, bias.reshape(1, HC), ln_w.reshape(1, HC),
      ln_b.reshape(1, HC))
    return out


# speedup vs baseline: 12.6819x; 1.1145x over previous
import functools

import jax
import jax.numpy as jnp
from jax import lax
from jax.experimental import pallas as pl
from jax.experimental.pallas import tpu as pltpu
from jax.experimental.pallas import tpu_sc as plsc

H = 4
C = 64
HC = H * C
ED = 16
AW = 128
G = 128
NTILES = 32


def _p1_body(x_ref, wl_ref, bl_ref, wr_ref, br_ref, xl_ref, xr_ref):
    xb = x_ref[...]
    xl_ref[...] = jnp.dot(xb, wl_ref[...], preferred_element_type=jnp.float32) + bl_ref[...]
    xr_ref[...] = jnp.dot(xb, wr_ref[...], preferred_element_type=jnp.float32) + br_ref[...]


def _make_gather(e_pad):
    n_chunks = e_pad // G
    mesh = plsc.VectorSubcoreMesh(core_axis_name="c", subcore_axis_name="s")

    @functools.partial(
        pl.kernel,
        mesh=mesh,
        out_type=jax.ShapeDtypeStruct((e_pad, HC), jnp.float32),
    )
    def gather_k(tbl_hbm, idx_hbm, o_hbm):
        def body(idx_vmem, o_vmem):
            pltpu.sync_copy(tbl_hbm.at[idx_vmem.at[0]], o_vmem)

        pltpu.emit_pipeline(
            body,
            grid=(n_chunks,),
            in_specs=[pl.BlockSpec((1, G), lambda i: (0, i))],
            out_specs=[pl.BlockSpec((G, HC), lambda i: (i, 0))],
            core_axis_name=("c", "s"),
            dimension_semantics=(pltpu.PARALLEL,),
        )(idx_hbm, o_hbm)

    return gather_k


def _p3_body(xl_ref, xr_ref, ea_ref, we_ref, att_ref, w1_ref, b1_ref,
             w2_ref, b2_ref, aux_ref, m_ref, m_acc):
    i = pl.program_id(0)
    ee = jnp.dot(ea_ref[...], we_ref[...], preferred_element_type=jnp.float32)
    v = xl_ref[...] + xr_ref[...] + ee
    v = jnp.where(v >= 0, v, 0.2 * v)
    vm = v * att_ref[...]
    cols = [jnp.sum(vm[:, h * C:(h + 1) * C], axis=1, keepdims=True)
            for h in range(H)]
    alpha = jnp.concatenate(cols, axis=1)
    g1 = jnp.dot(ea_ref[...], w1_ref[...], preferred_element_type=jnp.float32) + b1_ref[...]
    g1 = g1 * jax.nn.sigmoid(g1)
    g2 = jnp.dot(g1, w2_ref[...], preferred_element_type=jnp.float32) + b2_ref[...]
    gate = jax.nn.sigmoid(g2)
    one = jnp.ones_like(gate)
    zero = jnp.zeros_like(gate)
    aux_ref[...] = jnp.concatenate([alpha, gate, one, zero, zero], axis=1)
    blkmax = jnp.max(alpha)

    @pl.when(i == 0)
    def _():
        m_acc[0, 0] = blkmax

    @pl.when(i > 0)
    def _():
        m_acc[0, 0] = jnp.maximum(m_acc[0, 0], blkmax)

    m_ref[...] = jnp.full((1, 1), m_acc[0, 0], jnp.float32)


def _p4_body(e_real, be, aux_ref, xl_ref, m_ref, msg0_ref, msg1_ref, den_ref):
    i = pl.program_id(0)
    mglob = m_ref[...]
    aux = aux_ref[...]
    xl = xl_ref[...]
    rows = i * be + lax.broadcasted_iota(jnp.int32, (be, 1), 0)
    valid = (rows < e_real).astype(jnp.float32)
    a = jnp.exp(aux[:, 0:4] - mglob) * valid
    gate = aux[:, 4:5] * valid
    one = aux[:, 5:6] * valid
    msg0_ref[...] = jnp.concatenate(
        [xl[:, 0:64] * a[:, 0:1], xl[:, 64:128] * a[:, 1:2]], axis=1)
    msg1_ref[...] = jnp.concatenate(
        [xl[:, 128:192] * a[:, 2:3], xl[:, 192:256] * a[:, 3:4]], axis=1)
    den_ref[...] = jnp.concatenate(
        [a, gate, one, jnp.zeros((be, AW - 6), jnp.float32)], axis=1)


def _scatter_pipe(table_sh, dst2_hbm, msg_hbm, n_chunks, chunk0):
    def body(idx_vmem, msg_vmem):
        pltpu.sync_copy(msg_vmem, table_sh.at[idx_vmem.at[0]], add=True)

    pltpu.emit_pipeline(
        body,
        grid=(n_chunks,),
        in_specs=[pl.BlockSpec((1, G), lambda i: (0, i + chunk0)),
                  pl.BlockSpec((G, AW), lambda i: (i + chunk0, 0))],
        core_axis_name="s",
        dimension_semantics=(pltpu.PARALLEL,),
    )(dst2_hbm, msg_hbm)


def _make_scatter(n_nodes, e_pad):
    n_chunks = e_pad // G
    mesh = plsc.VectorSubcoreMesh(core_axis_name="c", subcore_axis_name="s")

    @functools.partial(
        pl.kernel,
        mesh=mesh,
        out_type=[jax.ShapeDtypeStruct((n_nodes, AW), jnp.float32),
                  jax.ShapeDtypeStruct((n_nodes, AW), jnp.float32)],
        scratch_types=[pltpu.VMEM_SHARED((n_nodes, AW), jnp.float32)],
    )
    def scatter_k(msg0_hbm, msg1_hbm, dst2_hbm, zeros_hbm, o0_hbm, o1_hbm,
                  table_sh):
        cid = lax.axis_index("c")
        sid = lax.axis_index("s")

        @pl.when(sid == 0)
        def _():
            pltpu.sync_copy(zeros_hbm, table_sh)

        plsc.subcore_barrier()

        @pl.when(cid == 0)
        def _():
            _scatter_pipe(table_sh, dst2_hbm, msg0_hbm, n_chunks, 0)

        @pl.when(cid == 1)
        def _():
            _scatter_pipe(table_sh, dst2_hbm, msg1_hbm, n_chunks, 0)

        plsc.subcore_barrier()

        @pl.when((sid == 0) & (cid == 0))
        def _():
            pltpu.sync_copy(table_sh, o0_hbm)

        @pl.when((sid == 0) & (cid == 1))
        def _():
            pltpu.sync_copy(table_sh, o1_hbm)

    return scatter_k


def _make_den_scatter(n_nodes, e_pad):
    half_chunks = e_pad // (2 * G)
    mesh = plsc.VectorSubcoreMesh(core_axis_name="c", subcore_axis_name="s")

    @functools.partial(
        pl.kernel,
        mesh=mesh,
        out_type=[jax.ShapeDtypeStruct((n_nodes, AW), jnp.float32),
                  jax.ShapeDtypeStruct((n_nodes, AW), jnp.float32)],
        scratch_types=[pltpu.VMEM_SHARED((n_nodes, AW), jnp.float32)],
    )
    def den_k(den_hbm, dst2_hbm, zeros_hbm, o0_hbm, o1_hbm, table_sh):
        cid = lax.axis_index("c")
        sid = lax.axis_index("s")

        @pl.when(sid == 0)
        def _():
            pltpu.sync_copy(zeros_hbm, table_sh)

        plsc.subcore_barrier()

        @pl.when(cid == 0)
        def _():
            _scatter_pipe(table_sh, dst2_hbm, den_hbm, half_chunks, 0)

        @pl.when(cid == 1)
        def _():
            _scatter_pipe(table_sh, dst2_hbm, den_hbm, half_chunks, half_chunks)

        plsc.subcore_barrier()

        @pl.when((sid == 0) & (cid == 0))
        def _():
            pltpu.sync_copy(table_sh, o0_hbm)

        @pl.when((sid == 0) & (cid == 1))
        def _():
            pltpu.sync_copy(table_sh, o1_hbm)

    return den_k


def _p6_body(acc0_ref, acc1_ref, den0_ref, den1_ref, x_ref, bias_ref,
             lnw_ref, lnb_ref, o_ref):
    a0 = acc0_ref[...]
    a1 = acc1_ref[...]
    bn = a0.shape[0]
    d8 = den0_ref[:, 0:8] + den1_ref[:, 0:8]
    num = jnp.concatenate([a0, a1], axis=1)
    den = jnp.concatenate(
        [jnp.broadcast_to(d8[:, h:h + 1], (bn, C)) for h in range(H)], axis=1)
    out = num / (den + 1e-30) + bias_ref[...]
    mean_gate = d8[:, 4:5] / jnp.maximum(d8[:, 5:6], 1.0)
    out = out * mean_gate
    mu = jnp.mean(out, axis=-1, keepdims=True)
    var = jnp.mean((out - mu) ** 2, axis=-1, keepdims=True)
    out = (out - mu) * lax.rsqrt(var + 1e-5) * lnw_ref[...] + lnb_ref[...]
    out = out * jax.nn.sigmoid(out)
    o_ref[...] = out + x_ref[...]


def kernel(x, edge_index, edge_attr, W_l, b_l, W_r, b_r, W_e, att, bias,
           eg_W1, eg_b1, eg_W2, eg_b2, ln_w, ln_b):
    n = x.shape[0]
    e = edge_attr.shape[0]
    e_pad = ((e + NTILES * G - 1) // (NTILES * G)) * (NTILES * G)
    pad = e_pad - e
    src_p = jnp.concatenate([edge_index[0].astype(jnp.int32),
                             jnp.zeros((pad,), jnp.int32)])
    dst_p = jnp.concatenate([edge_index[1].astype(jnp.int32),
                             jnp.zeros((pad,), jnp.int32)])
    ea_p = jnp.concatenate([edge_attr, jnp.zeros((pad, ED), edge_attr.dtype)])

    BN = 1000
    xl, xr = pl.pallas_call(
        _p1_body,
        out_shape=[jax.ShapeDtypeStruct((n, HC), jnp.float32)] * 2,
        grid=(n // BN,),
        in_specs=[
            pl.BlockSpec((BN, HC), lambda i: (i, 0)),
            pl.BlockSpec((HC, HC), lambda i: (0, 0)),
            pl.BlockSpec((1, HC), lambda i: (0, 0)),
            pl.BlockSpec((HC, HC), lambda i: (0, 0)),
            pl.BlockSpec((1, HC), lambda i: (0, 0)),
        ],
        out_specs=[pl.BlockSpec((BN, HC), lambda i: (i, 0))] * 2,
    )(x, W_l, b_l.reshape(1, HC), W_r, b_r.reshape(1, HC))

    gather = _make_gather(e_pad)
    xl_src = gather(xl, src_p.reshape(1, e_pad))
    xr_dst = gather(xr, dst_p.reshape(1, e_pad))

    BE = 2048
    n_eblk = e_pad // BE
    aux, mglob = pl.pallas_call(
        _p3_body,
        out_shape=[jax.ShapeDtypeStruct((e_pad, 8), jnp.float32),
                   jax.ShapeDtypeStruct((1, 1), jnp.float32)],
        grid=(n_eblk,),
        in_specs=[
            pl.BlockSpec((BE, HC), lambda i: (i, 0)),
            pl.BlockSpec((BE, HC), lambda i: (i, 0)),
            pl.BlockSpec((BE, ED), lambda i: (i, 0)),
            pl.BlockSpec((ED, HC), lambda i: (0, 0)),
            pl.BlockSpec((1, HC), lambda i: (0, 0)),
            pl.BlockSpec((ED, 2 * ED), lambda i: (0, 0)),
            pl.BlockSpec((1, 2 * ED), lambda i: (0, 0)),
            pl.BlockSpec((2 * ED, 1), lambda i: (0, 0)),
            pl.BlockSpec((1, 1), lambda i: (0, 0)),
        ],
        out_specs=[pl.BlockSpec((BE, 8), lambda i: (i, 0)),
                   pl.BlockSpec((1, 1), lambda i: (0, 0))],
        scratch_shapes=[pltpu.SMEM((1, 1), jnp.float32)],
    )(xl_src, xr_dst, ea_p, W_e, att.reshape(1, HC), eg_W1,
      eg_b1.reshape(1, 2 * ED), eg_W2, eg_b2.reshape(1, 1))

    msg0, msg1, denrows = pl.pallas_call(
        functools.partial(_p4_body, e, BE),
        out_shape=[jax.ShapeDtypeStruct((e_pad, AW), jnp.float32)] * 3,
        grid=(n_eblk,),
        in_specs=[
            pl.BlockSpec((BE, 8), lambda i: (i, 0)),
            pl.BlockSpec((BE, HC), lambda i: (i, 0)),
            pl.BlockSpec((1, 1), lambda i: (0, 0)),
        ],
        out_specs=[pl.BlockSpec((BE, AW), lambda i: (i, 0))] * 3,
    )(aux, xl_src, mglob)

    zeros_tbl = jnp.zeros((n, AW), jnp.float32)
    dst2 = dst_p.reshape(1, e_pad)
    acc0, acc1 = _make_scatter(n, e_pad)(msg0, msg1, dst2, zeros_tbl)
    den0, den1 = _make_den_scatter(n, e_pad)(denrows, dst2, zeros_tbl)

    out = pl.pallas_call(
        _p6_body,
        out_shape=jax.ShapeDtypeStruct((n, HC), jnp.float32),
        grid=(n // BN,),
        in_specs=[
            pl.BlockSpec((BN, AW), lambda i: (i, 0)),
            pl.BlockSpec((BN, AW), lambda i: (i, 0)),
            pl.BlockSpec((BN, AW), lambda i: (i, 0)),
            pl.BlockSpec((BN, AW), lambda i: (i, 0)),
            pl.BlockSpec((BN, HC), lambda i: (i, 0)),
            pl.BlockSpec((1, HC), lambda i: (0, 0)),
            pl.BlockSpec((1, HC), lambda i: (0, 0)),
            pl.BlockSpec((1, HC), lambda i: (0, 0)),
        ],
        out_specs=pl.BlockSpec((BN, HC), lambda i: (i, 0)),
    )(acc0, acc1, den0, den1, x, bias.reshape(1, HC), ln_w.reshape(1, HC),
      ln_b.reshape(1, HC))
    return out


# speedup vs baseline: 15.1030x; 1.1909x over previous
import functools

import jax
import jax.numpy as jnp
from jax import lax
from jax.experimental import pallas as pl
from jax.experimental.pallas import tpu as pltpu
from jax.experimental.pallas import tpu_sc as plsc

H = 4
C = 64
HC = H * C
ED = 16
AW = 128
G = 128
NTILES = 32


def _p1_body(x_ref, wl_ref, bl_ref, wr_ref, br_ref, xl_ref, xr_ref):
    xb = x_ref[...]
    xl_ref[...] = jnp.dot(xb, wl_ref[...], preferred_element_type=jnp.float32) + bl_ref[...]
    xr_ref[...] = jnp.dot(xb, wr_ref[...], preferred_element_type=jnp.float32) + br_ref[...]


def _make_gather(e_pad):
    n_chunks = e_pad // G
    mesh = plsc.VectorSubcoreMesh(core_axis_name="c", subcore_axis_name="s")

    @functools.partial(
        pl.kernel,
        mesh=mesh,
        out_type=jax.ShapeDtypeStruct((e_pad, HC), jnp.float32),
    )
    def gather_k(tbl_hbm, idx_hbm, o_hbm):
        def body(idx_vmem, o_vmem):
            pltpu.sync_copy(tbl_hbm.at[idx_vmem.at[0]], o_vmem)

        pltpu.emit_pipeline(
            body,
            grid=(n_chunks,),
            in_specs=[pl.BlockSpec((1, G), lambda i: (0, i))],
            out_specs=[pl.BlockSpec((G, HC), lambda i: (i, 0))],
            core_axis_name=("c", "s"),
            dimension_semantics=(pltpu.PARALLEL,),
        )(idx_hbm, o_hbm)

    return gather_k


def _p3_body(xl_ref, xr_ref, ea_ref, we_ref, attbd_ref, w1_ref, b1_ref,
             w2_ref, b2_ref, aux_ref, m_ref, m_acc):
    i = pl.program_id(0)
    ea = ea_ref[...]
    ee = jnp.dot(ea.astype(jnp.bfloat16), we_ref[...],
                 preferred_element_type=jnp.float32)
    v = xl_ref[...] + xr_ref[...] + ee
    v = jnp.where(v >= 0, v, 0.2 * v)
    alpha8 = jnp.dot(v.astype(jnp.bfloat16), attbd_ref[...],
                     preferred_element_type=jnp.float32)
    alpha = alpha8[:, 0:4]
    g1 = jnp.dot(ea.astype(jnp.bfloat16), w1_ref[...],
                 preferred_element_type=jnp.float32) + b1_ref[...]
    g1 = g1 * jax.nn.sigmoid(g1)
    g2 = jnp.sum(g1 * w2_ref[...], axis=1, keepdims=True) + b2_ref[...]
    gate = jax.nn.sigmoid(g2)
    one = jnp.ones_like(gate)
    zero = jnp.zeros_like(gate)
    aux_ref[...] = jnp.concatenate([alpha, gate, one, zero, zero], axis=1)
    blkmax = jnp.max(alpha)

    @pl.when(i == 0)
    def _():
        m_acc[0, 0] = blkmax

    @pl.when(i > 0)
    def _():
        m_acc[0, 0] = jnp.maximum(m_acc[0, 0], blkmax)

    m_ref[...] = jnp.full((1, 1), m_acc[0, 0], jnp.float32)


def _p4_body(e_real, be, aux_ref, xl_ref, m_ref, msg0_ref, msg1_ref, den_ref):
    i = pl.program_id(0)
    mglob = m_ref[...]
    aux = aux_ref[...]
    xl = xl_ref[...]
    rows = i * be + lax.broadcasted_iota(jnp.int32, (be, 1), 0)
    valid = (rows < e_real).astype(jnp.float32)
    a = jnp.exp(aux[:, 0:4] - mglob) * valid
    gate = aux[:, 4:5] * valid
    one = aux[:, 5:6] * valid
    msg0_ref[...] = jnp.concatenate(
        [xl[:, 0:64] * a[:, 0:1], xl[:, 64:128] * a[:, 1:2]], axis=1)
    msg1_ref[...] = jnp.concatenate(
        [xl[:, 128:192] * a[:, 2:3], xl[:, 192:256] * a[:, 3:4]], axis=1)
    den_ref[...] = jnp.concatenate(
        [a, gate, one, jnp.zeros((be, AW - 6), jnp.float32)], axis=1)


def _scatter_pipe(table_sh, dst2_hbm, msg_hbm, n_chunks, chunk0):
    def body(idx_vmem, msg_vmem):
        pltpu.sync_copy(msg_vmem, table_sh.at[idx_vmem.at[0]], add=True)

    pltpu.emit_pipeline(
        body,
        grid=(n_chunks,),
        in_specs=[pl.BlockSpec((1, G), lambda i: (0, i + chunk0)),
                  pl.BlockSpec((G, AW), lambda i: (i + chunk0, 0))],
        core_axis_name="s",
        dimension_semantics=(pltpu.PARALLEL,),
    )(dst2_hbm, msg_hbm)


def _make_scatter(n_nodes, e_pad):
    n_chunks = e_pad // G
    mesh = plsc.VectorSubcoreMesh(core_axis_name="c", subcore_axis_name="s")

    @functools.partial(
        pl.kernel,
        mesh=mesh,
        out_type=[jax.ShapeDtypeStruct((n_nodes, AW), jnp.float32),
                  jax.ShapeDtypeStruct((n_nodes, AW), jnp.float32)],
        scratch_types=[pltpu.VMEM_SHARED((n_nodes, AW), jnp.float32)],
    )
    def scatter_k(msg0_hbm, msg1_hbm, dst2_hbm, zeros_hbm, o0_hbm, o1_hbm,
                  table_sh):
        cid = lax.axis_index("c")
        sid = lax.axis_index("s")

        @pl.when(sid == 0)
        def _():
            pltpu.sync_copy(zeros_hbm, table_sh)

        plsc.subcore_barrier()

        @pl.when(cid == 0)
        def _():
            _scatter_pipe(table_sh, dst2_hbm, msg0_hbm, n_chunks, 0)

        @pl.when(cid == 1)
        def _():
            _scatter_pipe(table_sh, dst2_hbm, msg1_hbm, n_chunks, 0)

        plsc.subcore_barrier()

        @pl.when((sid == 0) & (cid == 0))
        def _():
            pltpu.sync_copy(table_sh, o0_hbm)

        @pl.when((sid == 0) & (cid == 1))
        def _():
            pltpu.sync_copy(table_sh, o1_hbm)

    return scatter_k


def _make_den_scatter(n_nodes, e_pad):
    half_chunks = e_pad // (2 * G)
    mesh = plsc.VectorSubcoreMesh(core_axis_name="c", subcore_axis_name="s")

    @functools.partial(
        pl.kernel,
        mesh=mesh,
        out_type=[jax.ShapeDtypeStruct((n_nodes, AW), jnp.float32),
                  jax.ShapeDtypeStruct((n_nodes, AW), jnp.float32)],
        scratch_types=[pltpu.VMEM_SHARED((n_nodes, AW), jnp.float32)],
    )
    def den_k(den_hbm, dst2_hbm, zeros_hbm, o0_hbm, o1_hbm, table_sh):
        cid = lax.axis_index("c")
        sid = lax.axis_index("s")

        @pl.when(sid == 0)
        def _():
            pltpu.sync_copy(zeros_hbm, table_sh)

        plsc.subcore_barrier()

        @pl.when(cid == 0)
        def _():
            _scatter_pipe(table_sh, dst2_hbm, den_hbm, half_chunks, 0)

        @pl.when(cid == 1)
        def _():
            _scatter_pipe(table_sh, dst2_hbm, den_hbm, half_chunks, half_chunks)

        plsc.subcore_barrier()

        @pl.when((sid == 0) & (cid == 0))
        def _():
            pltpu.sync_copy(table_sh, o0_hbm)

        @pl.when((sid == 0) & (cid == 1))
        def _():
            pltpu.sync_copy(table_sh, o1_hbm)

    return den_k


def _p6_body(acc0_ref, acc1_ref, den0_ref, den1_ref, x_ref, bias_ref,
             lnw_ref, lnb_ref, o_ref):
    a0 = acc0_ref[...]
    a1 = acc1_ref[...]
    bn = a0.shape[0]
    d8 = den0_ref[:, 0:8] + den1_ref[:, 0:8]
    num = jnp.concatenate([a0, a1], axis=1)
    den = jnp.concatenate(
        [jnp.broadcast_to(d8[:, h:h + 1], (bn, C)) for h in range(H)], axis=1)
    out = num / (den + 1e-30) + bias_ref[...]
    mean_gate = d8[:, 4:5] / jnp.maximum(d8[:, 5:6], 1.0)
    out = out * mean_gate
    mu = jnp.mean(out, axis=-1, keepdims=True)
    var = jnp.mean((out - mu) ** 2, axis=-1, keepdims=True)
    out = (out - mu) * lax.rsqrt(var + 1e-5) * lnw_ref[...] + lnb_ref[...]
    out = out * jax.nn.sigmoid(out)
    o_ref[...] = out + x_ref[...]


def kernel(x, edge_index, edge_attr, W_l, b_l, W_r, b_r, W_e, att, bias,
           eg_W1, eg_b1, eg_W2, eg_b2, ln_w, ln_b):
    n = x.shape[0]
    e = edge_attr.shape[0]
    e_pad = ((e + NTILES * G - 1) // (NTILES * G)) * (NTILES * G)
    pad = e_pad - e
    src_p = jnp.concatenate([edge_index[0].astype(jnp.int32),
                             jnp.zeros((pad,), jnp.int32)])
    dst_p = jnp.concatenate([edge_index[1].astype(jnp.int32),
                             jnp.zeros((pad,), jnp.int32)])
    ea_p = jnp.concatenate([edge_attr, jnp.zeros((pad, ED), edge_attr.dtype)])

    BN = 1000
    xl, xr = pl.pallas_call(
        _p1_body,
        out_shape=[jax.ShapeDtypeStruct((n, HC), jnp.float32)] * 2,
        grid=(n // BN,),
        in_specs=[
            pl.BlockSpec((BN, HC), lambda i: (i, 0)),
            pl.BlockSpec((HC, HC), lambda i: (0, 0)),
            pl.BlockSpec((1, HC), lambda i: (0, 0)),
            pl.BlockSpec((HC, HC), lambda i: (0, 0)),
            pl.BlockSpec((1, HC), lambda i: (0, 0)),
        ],
        out_specs=[pl.BlockSpec((BN, HC), lambda i: (i, 0))] * 2,
    )(x, W_l, b_l.reshape(1, HC), W_r, b_r.reshape(1, HC))

    gather = _make_gather(e_pad)
    xl_src = gather(xl, src_p.reshape(1, e_pad))
    xr_dst = gather(xr, dst_p.reshape(1, e_pad))

    BE = 2048
    n_eblk = e_pad // BE
    idx256 = jnp.arange(HC)
    attbd = jnp.where(idx256[:, None] // C == jnp.arange(8)[None, :],
                      att.reshape(HC)[:, None], 0.0).astype(jnp.bfloat16)
    aux, mglob = pl.pallas_call(
        _p3_body,
        out_shape=[jax.ShapeDtypeStruct((e_pad, 8), jnp.float32),
                   jax.ShapeDtypeStruct((1, 1), jnp.float32)],
        grid=(n_eblk,),
        in_specs=[
            pl.BlockSpec((BE, HC), lambda i: (i, 0)),
            pl.BlockSpec((BE, HC), lambda i: (i, 0)),
            pl.BlockSpec((BE, ED), lambda i: (i, 0)),
            pl.BlockSpec((ED, HC), lambda i: (0, 0)),
            pl.BlockSpec((HC, 8), lambda i: (0, 0)),
            pl.BlockSpec((ED, 2 * ED), lambda i: (0, 0)),
            pl.BlockSpec((1, 2 * ED), lambda i: (0, 0)),
            pl.BlockSpec((1, 2 * ED), lambda i: (0, 0)),
            pl.BlockSpec((1, 1), lambda i: (0, 0)),
        ],
        out_specs=[pl.BlockSpec((BE, 8), lambda i: (i, 0)),
                   pl.BlockSpec((1, 1), lambda i: (0, 0))],
        scratch_shapes=[pltpu.SMEM((1, 1), jnp.float32)],
    )(xl_src, xr_dst, ea_p, W_e.astype(jnp.bfloat16), attbd,
      eg_W1.astype(jnp.bfloat16), eg_b1.reshape(1, 2 * ED),
      eg_W2.reshape(1, 2 * ED), eg_b2.reshape(1, 1))

    msg0, msg1, denrows = pl.pallas_call(
        functools.partial(_p4_body, e, BE),
        out_shape=[jax.ShapeDtypeStruct((e_pad, AW), jnp.float32)] * 3,
        grid=(n_eblk,),
        in_specs=[
            pl.BlockSpec((BE, 8), lambda i: (i, 0)),
            pl.BlockSpec((BE, HC), lambda i: (i, 0)),
            pl.BlockSpec((1, 1), lambda i: (0, 0)),
        ],
        out_specs=[pl.BlockSpec((BE, AW), lambda i: (i, 0))] * 3,
    )(aux, xl_src, mglob)

    zeros_tbl = jnp.zeros((n, AW), jnp.float32)
    dst2 = dst_p.reshape(1, e_pad)
    acc0, acc1 = _make_scatter(n, e_pad)(msg0, msg1, dst2, zeros_tbl)
    den0, den1 = _make_den_scatter(n, e_pad)(denrows, dst2, zeros_tbl)

    out = pl.pallas_call(
        _p6_body,
        out_shape=jax.ShapeDtypeStruct((n, HC), jnp.float32),
        grid=(n // BN,),
        in_specs=[
            pl.BlockSpec((BN, AW), lambda i: (i, 0)),
            pl.BlockSpec((BN, AW), lambda i: (i, 0)),
            pl.BlockSpec((BN, AW), lambda i: (i, 0)),
            pl.BlockSpec((BN, AW), lambda i: (i, 0)),
            pl.BlockSpec((BN, HC), lambda i: (i, 0)),
            pl.BlockSpec((1, HC), lambda i: (0, 0)),
            pl.BlockSpec((1, HC), lambda i: (0, 0)),
            pl.BlockSpec((1, HC), lambda i: (0, 0)),
        ],
        out_specs=pl.BlockSpec((BN, HC), lambda i: (i, 0)),
    )(acc0, acc1, den0, den1, x, bias.reshape(1, HC), ln_w.reshape(1, HC),
      ln_b.reshape(1, HC))
    return out


# speedup vs baseline: 15.2433x; 1.0093x over previous
import functools

import jax
import jax.numpy as jnp
from jax import lax
from jax.experimental import pallas as pl
from jax.experimental.pallas import tpu as pltpu
from jax.experimental.pallas import tpu_sc as plsc

H = 4
C = 64
HC = H * C
ED = 16
AW = 128
G = 128
NTILES = 32


def _p1_body(x_ref, wl_ref, bl_ref, wr_ref, br_ref, xl_ref, xr_ref):
    xb = x_ref[...]
    xl_ref[...] = jnp.dot(xb, wl_ref[...], preferred_element_type=jnp.float32) + bl_ref[...]
    xr_ref[...] = jnp.dot(xb, wr_ref[...], preferred_element_type=jnp.float32) + br_ref[...]


def _make_gather(e_pad):
    n_chunks = e_pad // G
    mesh = plsc.VectorSubcoreMesh(core_axis_name="c", subcore_axis_name="s")

    c0_chunks = (n_chunks * 13 // 20) // 16 * 16

    @functools.partial(
        pl.kernel,
        mesh=mesh,
        out_type=jax.ShapeDtypeStruct((e_pad, HC), jnp.float32),
    )
    def gather_k(tbl_hbm, idx_hbm, o_hbm):
        cid = lax.axis_index("c")

        def pipe(n_ch, ch0):
            def body(idx_vmem, o_vmem):
                pltpu.sync_copy(tbl_hbm.at[idx_vmem.at[0]], o_vmem)

            pltpu.emit_pipeline(
                body,
                grid=(n_ch,),
                in_specs=[pl.BlockSpec((1, G), lambda i: (0, i + ch0))],
                out_specs=[pl.BlockSpec((G, HC), lambda i: (i + ch0, 0))],
                core_axis_name="s",
                dimension_semantics=(pltpu.PARALLEL,),
            )(idx_hbm, o_hbm)

        @pl.when(cid == 0)
        def _():
            pipe(c0_chunks, 0)

        @pl.when(cid == 1)
        def _():
            pipe(n_chunks - c0_chunks, c0_chunks)

    return gather_k


def _p3_body(xl_ref, xr_ref, ea_ref, we_ref, attbd_ref, w1_ref, b1_ref,
             w2_ref, b2_ref, aux_ref, m_ref, m_acc):
    i = pl.program_id(0)
    ea = ea_ref[...]
    ee = jnp.dot(ea.astype(jnp.bfloat16), we_ref[...],
                 preferred_element_type=jnp.float32)
    v = xl_ref[...] + xr_ref[...] + ee
    v = jnp.where(v >= 0, v, 0.2 * v)
    alpha8 = jnp.dot(v.astype(jnp.bfloat16), attbd_ref[...],
                     preferred_element_type=jnp.float32)
    alpha = alpha8[:, 0:4]
    g1 = jnp.dot(ea.astype(jnp.bfloat16), w1_ref[...],
                 preferred_element_type=jnp.float32) + b1_ref[...]
    g1 = g1 * jax.nn.sigmoid(g1)
    g2 = jnp.sum(g1 * w2_ref[...], axis=1, keepdims=True) + b2_ref[...]
    gate = jax.nn.sigmoid(g2)
    one = jnp.ones_like(gate)
    zero = jnp.zeros_like(gate)
    aux_ref[...] = jnp.concatenate([alpha, gate, one, zero, zero], axis=1)
    blkmax = jnp.max(alpha)

    @pl.when(i == 0)
    def _():
        m_acc[0, 0] = blkmax

    @pl.when(i > 0)
    def _():
        m_acc[0, 0] = jnp.maximum(m_acc[0, 0], blkmax)

    m_ref[...] = jnp.full((1, 1), m_acc[0, 0], jnp.float32)


def _p4_body(e_real, be, aux_ref, xl_ref, m_ref, msg0_ref, msg1_ref, den_ref):
    i = pl.program_id(0)
    mglob = m_ref[...]
    aux = aux_ref[...]
    xl = xl_ref[...]
    rows = i * be + lax.broadcasted_iota(jnp.int32, (be, 1), 0)
    valid = (rows < e_real).astype(jnp.float32)
    a = jnp.exp(aux[:, 0:4] - mglob) * valid
    gate = aux[:, 4:5] * valid
    one = aux[:, 5:6] * valid
    msg0_ref[...] = jnp.concatenate(
        [xl[:, 0:64] * a[:, 0:1], xl[:, 64:128] * a[:, 1:2]], axis=1)
    msg1_ref[...] = jnp.concatenate(
        [xl[:, 128:192] * a[:, 2:3], xl[:, 192:256] * a[:, 3:4]], axis=1)
    den_ref[...] = jnp.concatenate(
        [a, gate, one, jnp.zeros((be, AW - 6), jnp.float32)], axis=1)


def _scatter_pipe(table_sh, dst2_hbm, msg_hbm, n_chunks, chunk0):
    def body(idx_vmem, msg_vmem):
        pltpu.sync_copy(msg_vmem, table_sh.at[idx_vmem.at[0]], add=True)

    pltpu.emit_pipeline(
        body,
        grid=(n_chunks,),
        in_specs=[pl.BlockSpec((1, G), lambda i: (0, i + chunk0)),
                  pl.BlockSpec((G, AW), lambda i: (i + chunk0, 0))],
        core_axis_name="s",
        dimension_semantics=(pltpu.PARALLEL,),
    )(dst2_hbm, msg_hbm)


def _make_scatter(n_nodes, e_pad):
    n_chunks = e_pad // G
    mesh = plsc.VectorSubcoreMesh(core_axis_name="c", subcore_axis_name="s")

    @functools.partial(
        pl.kernel,
        mesh=mesh,
        out_type=[jax.ShapeDtypeStruct((n_nodes, AW), jnp.float32),
                  jax.ShapeDtypeStruct((n_nodes, AW), jnp.float32)],
        scratch_types=[pltpu.VMEM_SHARED((n_nodes, AW), jnp.float32)],
    )
    def scatter_k(msg0_hbm, msg1_hbm, dst2_hbm, zeros_hbm, o0_hbm, o1_hbm,
                  table_sh):
        cid = lax.axis_index("c")
        sid = lax.axis_index("s")

        @pl.when(sid == 0)
        def _():
            pltpu.sync_copy(zeros_hbm, table_sh)

        plsc.subcore_barrier()

        @pl.when(cid == 0)
        def _():
            _scatter_pipe(table_sh, dst2_hbm, msg0_hbm, n_chunks, 0)

        @pl.when(cid == 1)
        def _():
            _scatter_pipe(table_sh, dst2_hbm, msg1_hbm, n_chunks, 0)

        plsc.subcore_barrier()

        @pl.when((sid == 0) & (cid == 0))
        def _():
            pltpu.sync_copy(table_sh, o0_hbm)

        @pl.when((sid == 0) & (cid == 1))
        def _():
            pltpu.sync_copy(table_sh, o1_hbm)

    return scatter_k


def _make_den_scatter(n_nodes, e_pad):
    half_chunks = e_pad // (2 * G)
    mesh = plsc.VectorSubcoreMesh(core_axis_name="c", subcore_axis_name="s")

    @functools.partial(
        pl.kernel,
        mesh=mesh,
        out_type=[jax.ShapeDtypeStruct((n_nodes, AW), jnp.float32),
                  jax.ShapeDtypeStruct((n_nodes, AW), jnp.float32)],
        scratch_types=[pltpu.VMEM_SHARED((n_nodes, AW), jnp.float32)],
    )
    def den_k(den_hbm, dst2_hbm, zeros_hbm, o0_hbm, o1_hbm, table_sh):
        cid = lax.axis_index("c")
        sid = lax.axis_index("s")

        @pl.when(sid == 0)
        def _():
            pltpu.sync_copy(zeros_hbm, table_sh)

        plsc.subcore_barrier()

        @pl.when(cid == 0)
        def _():
            _scatter_pipe(table_sh, dst2_hbm, den_hbm, half_chunks, 0)

        @pl.when(cid == 1)
        def _():
            _scatter_pipe(table_sh, dst2_hbm, den_hbm, half_chunks, half_chunks)

        plsc.subcore_barrier()

        @pl.when((sid == 0) & (cid == 0))
        def _():
            pltpu.sync_copy(table_sh, o0_hbm)

        @pl.when((sid == 0) & (cid == 1))
        def _():
            pltpu.sync_copy(table_sh, o1_hbm)

    return den_k


def _p6_body(acc0_ref, acc1_ref, den0_ref, den1_ref, x_ref, bias_ref,
             lnw_ref, lnb_ref, o_ref):
    a0 = acc0_ref[...]
    a1 = acc1_ref[...]
    bn = a0.shape[0]
    d8 = den0_ref[:, 0:8] + den1_ref[:, 0:8]
    num = jnp.concatenate([a0, a1], axis=1)
    den = jnp.concatenate(
        [jnp.broadcast_to(d8[:, h:h + 1], (bn, C)) for h in range(H)], axis=1)
    out = num / (den + 1e-30) + bias_ref[...]
    mean_gate = d8[:, 4:5] / jnp.maximum(d8[:, 5:6], 1.0)
    out = out * mean_gate
    mu = jnp.mean(out, axis=-1, keepdims=True)
    var = jnp.mean((out - mu) ** 2, axis=-1, keepdims=True)
    out = (out - mu) * lax.rsqrt(var + 1e-5) * lnw_ref[...] + lnb_ref[...]
    out = out * jax.nn.sigmoid(out)
    o_ref[...] = out + x_ref[...]


def kernel(x, edge_index, edge_attr, W_l, b_l, W_r, b_r, W_e, att, bias,
           eg_W1, eg_b1, eg_W2, eg_b2, ln_w, ln_b):
    n = x.shape[0]
    e = edge_attr.shape[0]
    e_pad = ((e + NTILES * G - 1) // (NTILES * G)) * (NTILES * G)
    pad = e_pad - e
    src_p = jnp.concatenate([edge_index[0].astype(jnp.int32),
                             jnp.zeros((pad,), jnp.int32)])
    dst_p = jnp.concatenate([edge_index[1].astype(jnp.int32),
                             jnp.zeros((pad,), jnp.int32)])
    ea_p = jnp.concatenate([edge_attr, jnp.zeros((pad, ED), edge_attr.dtype)])

    BN = 1000
    xl, xr = pl.pallas_call(
        _p1_body,
        out_shape=[jax.ShapeDtypeStruct((n, HC), jnp.float32)] * 2,
        grid=(n // BN,),
        in_specs=[
            pl.BlockSpec((BN, HC), lambda i: (i, 0)),
            pl.BlockSpec((HC, HC), lambda i: (0, 0)),
            pl.BlockSpec((1, HC), lambda i: (0, 0)),
            pl.BlockSpec((HC, HC), lambda i: (0, 0)),
            pl.BlockSpec((1, HC), lambda i: (0, 0)),
        ],
        out_specs=[pl.BlockSpec((BN, HC), lambda i: (i, 0))] * 2,
    )(x, W_l, b_l.reshape(1, HC), W_r, b_r.reshape(1, HC))

    gather = _make_gather(e_pad)
    xl_src = gather(xl, src_p.reshape(1, e_pad))
    xr_dst = gather(xr, dst_p.reshape(1, e_pad))

    BE = 2048
    n_eblk = e_pad // BE
    idx256 = jnp.arange(HC)
    attbd = jnp.where(idx256[:, None] // C == jnp.arange(8)[None, :],
                      att.reshape(HC)[:, None], 0.0).astype(jnp.bfloat16)
    aux, mglob = pl.pallas_call(
        _p3_body,
        out_shape=[jax.ShapeDtypeStruct((e_pad, 8), jnp.float32),
                   jax.ShapeDtypeStruct((1, 1), jnp.float32)],
        grid=(n_eblk,),
        in_specs=[
            pl.BlockSpec((BE, HC), lambda i: (i, 0)),
            pl.BlockSpec((BE, HC), lambda i: (i, 0)),
            pl.BlockSpec((BE, ED), lambda i: (i, 0)),
            pl.BlockSpec((ED, HC), lambda i: (0, 0)),
            pl.BlockSpec((HC, 8), lambda i: (0, 0)),
            pl.BlockSpec((ED, 2 * ED), lambda i: (0, 0)),
            pl.BlockSpec((1, 2 * ED), lambda i: (0, 0)),
            pl.BlockSpec((1, 2 * ED), lambda i: (0, 0)),
            pl.BlockSpec((1, 1), lambda i: (0, 0)),
        ],
        out_specs=[pl.BlockSpec((BE, 8), lambda i: (i, 0)),
                   pl.BlockSpec((1, 1), lambda i: (0, 0))],
        scratch_shapes=[pltpu.SMEM((1, 1), jnp.float32)],
    )(xl_src, xr_dst, ea_p, W_e.astype(jnp.bfloat16), attbd,
      eg_W1.astype(jnp.bfloat16), eg_b1.reshape(1, 2 * ED),
      eg_W2.reshape(1, 2 * ED), eg_b2.reshape(1, 1))

    msg0, msg1, denrows = pl.pallas_call(
        functools.partial(_p4_body, e, BE),
        out_shape=[jax.ShapeDtypeStruct((e_pad, AW), jnp.float32)] * 3,
        grid=(n_eblk,),
        in_specs=[
            pl.BlockSpec((BE, 8), lambda i: (i, 0)),
            pl.BlockSpec((BE, HC), lambda i: (i, 0)),
            pl.BlockSpec((1, 1), lambda i: (0, 0)),
        ],
        out_specs=[pl.BlockSpec((BE, AW), lambda i: (i, 0))] * 3,
    )(aux, xl_src, mglob)

    zeros_tbl = jnp.zeros((n, AW), jnp.float32)
    dst2 = dst_p.reshape(1, e_pad)
    acc0, acc1 = _make_scatter(n, e_pad)(msg0, msg1, dst2, zeros_tbl)
    den0, den1 = _make_den_scatter(n, e_pad)(denrows, dst2, zeros_tbl)

    out = pl.pallas_call(
        _p6_body,
        out_shape=jax.ShapeDtypeStruct((n, HC), jnp.float32),
        grid=(n // BN,),
        in_specs=[
            pl.BlockSpec((BN, AW), lambda i: (i, 0)),
            pl.BlockSpec((BN, AW), lambda i: (i, 0)),
            pl.BlockSpec((BN, AW), lambda i: (i, 0)),
            pl.BlockSpec((BN, AW), lambda i: (i, 0)),
            pl.BlockSpec((BN, HC), lambda i: (i, 0)),
            pl.BlockSpec((1, HC), lambda i: (0, 0)),
            pl.BlockSpec((1, HC), lambda i: (0, 0)),
            pl.BlockSpec((1, HC), lambda i: (0, 0)),
        ],
        out_specs=pl.BlockSpec((BN, HC), lambda i: (i, 0)),
    )(acc0, acc1, den0, den1, x, bias.reshape(1, HC), ln_w.reshape(1, HC),
      ln_b.reshape(1, HC))
    return out
